# Initial kernel scaffold; baseline (speedup 1.0000x reference)
import jax
import jax.numpy as jnp
from jax.experimental import pallas as pl


def kernel(token_ids, U, V):
    raise NotImplementedError("write your pallas kernel here")


# speedup vs baseline: 2.8453x; 2.8453x over previous
import functools

import jax
import jax.numpy as jnp
from jax import lax
from jax.experimental import pallas as pl
from jax.experimental.pallas import tpu as pltpu
from jax.experimental.pallas import tpu_sc as plsc

DIM = 128
RANK = 64

NC = 2
NS = 16
NW = NC * NS


def _mm_body(u_ref, v_ref, e_ref):
    e_ref[...] = lax.dot_general(
        u_ref[...], v_ref[...],
        dimension_numbers=(((1,), (0,)), ((), ())),
        preferred_element_type=jnp.float32,
        precision=lax.Precision.HIGHEST,
    )


@functools.partial(jax.jit, static_argnames=("blk",))
def _compute_table(U, V, blk=1000):
    vocab = U.shape[0]
    return pl.pallas_call(
        _mm_body,
        grid=(vocab // blk,),
        in_specs=[
            pl.BlockSpec((blk, RANK), lambda i: (i, 0)),
            pl.BlockSpec((RANK, DIM), lambda i: (0, 0)),
        ],
        out_specs=pl.BlockSpec((blk, DIM), lambda i: (i, 0)),
        out_shape=jax.ShapeDtypeStruct((vocab, DIM), jnp.float32),
    )(U, V)


@functools.lru_cache(maxsize=None)
def _make_gather(n_tokens, chunk):
    per_w = n_tokens // NW
    n_chunks = per_w // chunk
    assert per_w % chunk == 0 and n_tokens % NW == 0

    mesh = plsc.VectorSubcoreMesh(core_axis_name="c", subcore_axis_name="s")

    @functools.partial(
        pl.kernel,
        out_type=jax.ShapeDtypeStruct((n_tokens, DIM), jnp.float32),
        mesh=mesh,
        scratch_types=[
            pltpu.VMEM((chunk,), jnp.int32),
            pltpu.VMEM((chunk, DIM), jnp.float32),
            pltpu.SemaphoreType.DMA,
        ],
    )
    def gather_kernel(table_hbm, idx_hbm, out_hbm, idx_v, rows_v, gsem):
        wid = lax.axis_index("s") * NC + lax.axis_index("c")
        base = wid * per_w

        def step(g, carry):
            off = base + g * chunk
            pltpu.sync_copy(idx_hbm.at[pl.ds(off, chunk)], idx_v)
            pltpu.async_copy(table_hbm.at[idx_v], rows_v, gsem).wait()
            pltpu.sync_copy(rows_v, out_hbm.at[pl.ds(off, chunk)])
            return carry

        lax.fori_loop(0, n_chunks, step, 0)

    return gather_kernel


def kernel(token_ids, U, V):
    B, L = token_ids.shape
    n_tokens = B * L
    ids = token_ids.reshape(n_tokens).astype(jnp.int32)
    table = _compute_table(U, V)
    out = _make_gather(n_tokens, 128)(table, ids)
    return out.reshape(B, L, DIM)


# speedup vs baseline: 3.5538x; 1.2490x over previous
import functools

import jax
import jax.numpy as jnp
from jax import lax
from jax.experimental import pallas as pl
from jax.experimental.pallas import tpu as pltpu
from jax.experimental.pallas import tpu_sc as plsc

DIM = 128
RANK = 64

NC = 2
NS = 16
NW = NC * NS


def _mm_body(u_ref, v_ref, e_ref):
    e_ref[...] = lax.dot_general(
        u_ref[...], v_ref[...],
        dimension_numbers=(((1,), (0,)), ((), ())),
        preferred_element_type=jnp.float32,
        precision=lax.Precision.HIGHEST,
    )


@functools.partial(jax.jit, static_argnames=("blk",))
def _compute_table(U, V, blk=1000):
    vocab = U.shape[0]
    return pl.pallas_call(
        _mm_body,
        grid=(vocab // blk,),
        in_specs=[
            pl.BlockSpec((blk, RANK), lambda i: (i, 0)),
            pl.BlockSpec((RANK, DIM), lambda i: (0, 0)),
        ],
        out_specs=pl.BlockSpec((blk, DIM), lambda i: (i, 0)),
        out_shape=jax.ShapeDtypeStruct((vocab, DIM), jnp.float32),
    )(U, V)


@functools.lru_cache(maxsize=None)
def _make_gather(n_tokens, chunk, nbuf):
    n_rows_idx = n_tokens // chunk
    n_chunks = n_rows_idx // NW
    assert n_tokens % (NW * chunk) == 0 and n_chunks % nbuf == 0
    n_outer = n_chunks // nbuf

    mesh = plsc.VectorSubcoreMesh(core_axis_name="c", subcore_axis_name="s")
    scratch = [
        pltpu.VMEM((n_chunks, chunk), jnp.int32),
        pltpu.VMEM((nbuf, chunk, DIM), jnp.float32),
    ]
    scratch += [pltpu.SemaphoreType.DMA] * (2 * nbuf)

    @functools.partial(
        pl.kernel,
        out_type=jax.ShapeDtypeStruct((n_tokens, DIM), jnp.float32),
        mesh=mesh,
        scratch_types=scratch,
    )
    def gather_kernel(table_hbm, idx_hbm, out_hbm, idx_v, rows_v, *sems):
        gs = sems[:nbuf]
        ws = sems[nbuf:]
        wid = lax.axis_index("s") * NC + lax.axis_index("c")
        rbase = wid * n_chunks
        tbase = rbase * chunk

        def start_gather(g, slot):
            pltpu.async_copy(table_hbm.at[idx_v.at[g]], rows_v.at[slot], gs[slot])

        def wait_gather(slot):
            pltpu.make_async_copy(
                table_hbm.at[idx_v.at[0]], rows_v.at[slot], gs[slot]
            ).wait()

        def start_write(g, slot):
            pltpu.async_copy(
                rows_v.at[slot], out_hbm.at[pl.ds(tbase + g * chunk, chunk)],
                ws[slot],
            )

        def wait_write(slot):
            pltpu.make_async_copy(
                rows_v.at[slot], out_hbm.at[pl.ds(tbase, chunk)], ws[slot]
            ).wait()

        pltpu.sync_copy(idx_hbm.at[wid], idx_v)
        for b in range(nbuf - 1):
            start_gather(b, b)

        def outer(j, carry):
            for b in range(nbuf):
                s = b
                sp = (b - 1) % nbuf
                g = j * nbuf + b
                gp = g + nbuf - 1
                if b == 0:
                    @pl.when(j > 0)
                    def _():
                        wait_write(sp)
                    start_gather(gp, sp)
                else:
                    @pl.when(j < n_outer - 1)
                    def _():
                        wait_write(sp)
                        start_gather(gp, sp)
                wait_gather(s)
                start_write(g, s)
            return carry

        lax.fori_loop(0, n_outer, outer, 0)
        for s in range(nbuf):
            wait_write(s)

    return gather_kernel


def kernel(token_ids, U, V):
    B, L = token_ids.shape
    n_tokens = B * L
    chunk = 128
    ids = token_ids.reshape(NW, n_tokens // (NW * chunk), chunk).astype(jnp.int32)
    table = _compute_table(U, V)
    out = _make_gather(n_tokens, chunk, 5)(table, ids)
    return out.reshape(B, L, DIM)


# speedup vs baseline: 4.7330x; 1.3318x over previous
import functools

import jax
import jax.numpy as jnp
from jax import lax
from jax.experimental import pallas as pl
from jax.experimental.pallas import tpu as pltpu
from jax.experimental.pallas import tpu_sc as plsc

DIM = 128
RANK = 64

NC = 2
NS = 16
NW = NC * NS


def _mm_body(u_ref, v_ref, e_ref):
    e_ref[...] = lax.dot_general(
        u_ref[...], v_ref[...],
        dimension_numbers=(((1,), (0,)), ((), ())),
        preferred_element_type=jnp.float32,
        precision=lax.Precision.DEFAULT,
    )


@functools.partial(jax.jit, static_argnames=("blk",))
def _compute_table(U, V, blk=2000):
    vocab = U.shape[0]
    return pl.pallas_call(
        _mm_body,
        grid=(vocab // blk,),
        in_specs=[
            pl.BlockSpec((blk, RANK), lambda i: (i, 0)),
            pl.BlockSpec((RANK, DIM), lambda i: (0, 0)),
        ],
        out_specs=pl.BlockSpec((blk, DIM), lambda i: (i, 0)),
        out_shape=jax.ShapeDtypeStruct((vocab, DIM), jnp.float32),
    )(U, V)


@functools.lru_cache(maxsize=None)
def _make_gather(n_tokens, chunk, nbuf):
    n_rows_idx = n_tokens // chunk
    n_chunks = n_rows_idx // NW
    assert n_tokens % (NW * chunk) == 0 and n_chunks % nbuf == 0
    n_outer = n_chunks // nbuf

    mesh = plsc.VectorSubcoreMesh(core_axis_name="c", subcore_axis_name="s")
    scratch = [
        pltpu.VMEM((n_chunks, chunk), jnp.int32),
        pltpu.VMEM((nbuf, chunk, DIM), jnp.float32),
    ]
    scratch += [pltpu.SemaphoreType.DMA] * (2 * nbuf)

    @functools.partial(
        pl.kernel,
        out_type=jax.ShapeDtypeStruct((n_tokens, DIM), jnp.float32),
        mesh=mesh,
        scratch_types=scratch,
    )
    def gather_kernel(table_hbm, idx_hbm, out_hbm, idx_v, rows_v, *sems):
        gs = sems[:nbuf]
        ws = sems[nbuf:]
        wid = lax.axis_index("s") * NC + lax.axis_index("c")
        rbase = wid * n_chunks
        tbase = rbase * chunk

        def start_gather(g, slot):
            pltpu.async_copy(table_hbm.at[idx_v.at[g]], rows_v.at[slot], gs[slot])

        def wait_gather(slot):
            pltpu.make_async_copy(
                table_hbm.at[idx_v.at[0]], rows_v.at[slot], gs[slot]
            ).wait()

        def start_write(g, slot):
            pltpu.async_copy(
                rows_v.at[slot], out_hbm.at[pl.ds(tbase + g * chunk, chunk)],
                ws[slot],
            )

        def wait_write(slot):
            pltpu.make_async_copy(
                rows_v.at[slot], out_hbm.at[pl.ds(tbase, chunk)], ws[slot]
            ).wait()

        pltpu.sync_copy(idx_hbm.at[wid], idx_v)
        for b in range(nbuf - 1):
            start_gather(b, b)

        def outer(j, carry):
            for b in range(nbuf):
                s = b
                sp = (b - 1) % nbuf
                g = j * nbuf + b
                gp = g + nbuf - 1
                if b == 0:
                    @pl.when(j > 0)
                    def _():
                        wait_write(sp)
                    start_gather(gp, sp)
                else:
                    @pl.when(j < n_outer - 1)
                    def _():
                        wait_write(sp)
                        start_gather(gp, sp)
                wait_gather(s)
                start_write(g, s)
            return carry

        lax.fori_loop(0, n_outer, outer, 0)
        for s in range(nbuf):
            wait_write(s)

    return gather_kernel


def kernel(token_ids, U, V):
    B, L = token_ids.shape
    n_tokens = B * L
    chunk = 128
    ids = token_ids.reshape(NW, n_tokens // (NW * chunk), chunk).astype(jnp.int32)
    table = _compute_table(U, V)
    out = _make_gather(n_tokens, chunk, 5)(table, ids)
    return out.reshape(B, L, DIM)


# speedup vs baseline: 5.1796x; 1.0944x over previous
import functools

import jax
import jax.numpy as jnp
from jax import lax
from jax.experimental import pallas as pl
from jax.experimental.pallas import tpu as pltpu
from jax.experimental.pallas import tpu_sc as plsc

DIM = 128
RANK = 64

NC = 2
NS = 16
NW = NC * NS


def _mm_body(u_ref, v_ref, e_ref):
    e_ref[...] = lax.dot_general(
        u_ref[...], v_ref[...],
        dimension_numbers=(((1,), (0,)), ((), ())),
        preferred_element_type=jnp.float32,
        precision=lax.Precision.DEFAULT,
    )


@functools.partial(jax.jit, static_argnames=("blk",))
def _compute_table(U, V, blk=5000):
    vocab = U.shape[0]
    return pl.pallas_call(
        _mm_body,
        grid=(vocab // blk,),
        in_specs=[
            pl.BlockSpec((blk, RANK), lambda i: (i, 0)),
            pl.BlockSpec((RANK, DIM), lambda i: (0, 0)),
        ],
        out_specs=pl.BlockSpec((blk, DIM), lambda i: (i, 0)),
        out_shape=jax.ShapeDtypeStruct((vocab, DIM), jnp.float32),
    )(U, V)


@functools.lru_cache(maxsize=None)
def _make_gather(n_tokens, chunk, nbuf):
    n_rows_idx = n_tokens // chunk
    n_chunks = n_rows_idx // NW
    assert n_tokens % (NW * chunk) == 0 and n_chunks % nbuf == 0
    n_outer = n_chunks // nbuf

    mesh = plsc.VectorSubcoreMesh(core_axis_name="c", subcore_axis_name="s")
    scratch = [
        pltpu.VMEM((n_chunks, chunk), jnp.int32),
        pltpu.VMEM((nbuf, chunk, DIM), jnp.float32),
    ]
    scratch += [pltpu.SemaphoreType.DMA] * (2 * nbuf)

    @functools.partial(
        pl.kernel,
        out_type=jax.ShapeDtypeStruct((n_tokens, DIM), jnp.float32),
        mesh=mesh,
        scratch_types=scratch,
    )
    def gather_kernel(table_hbm, idx_hbm, out_hbm, idx_v, rows_v, *sems):
        gs = sems[:nbuf]
        ws = sems[nbuf:]
        wid = lax.axis_index("s") * NC + lax.axis_index("c")
        rbase = wid * n_chunks
        tbase = rbase * chunk

        def start_gather(g, slot):
            pltpu.async_copy(table_hbm.at[idx_v.at[g]], rows_v.at[slot], gs[slot])

        def wait_gather(slot):
            pltpu.make_async_copy(
                table_hbm.at[idx_v.at[0]], rows_v.at[slot], gs[slot]
            ).wait()

        def start_write(g, slot):
            pltpu.async_copy(
                rows_v.at[slot], out_hbm.at[pl.ds(tbase + g * chunk, chunk)],
                ws[slot],
            )

        def wait_write(slot):
            pltpu.make_async_copy(
                rows_v.at[slot], out_hbm.at[pl.ds(tbase, chunk)], ws[slot]
            ).wait()

        pltpu.sync_copy(idx_hbm.at[wid], idx_v)
        for b in range(nbuf - 1):
            start_gather(b, b)

        def outer(j, carry):
            for b in range(nbuf):
                s = b
                sp = (b - 1) % nbuf
                g = j * nbuf + b
                gp = g + nbuf - 1
                if b == 0:
                    @pl.when(j > 0)
                    def _():
                        wait_write(sp)
                    start_gather(gp, sp)
                else:
                    @pl.when(j < n_outer - 1)
                    def _():
                        wait_write(sp)
                        start_gather(gp, sp)
                wait_gather(s)
                start_write(g, s)
            return carry

        lax.fori_loop(0, n_outer, outer, 0)
        for s in range(nbuf):
            wait_write(s)

    return gather_kernel


def kernel(token_ids, U, V):
    B, L = token_ids.shape
    n_tokens = B * L
    chunk = 128
    ids = token_ids.reshape(NW, n_tokens // (NW * chunk), chunk).astype(jnp.int32)
    table = _compute_table(U, V)
    out = _make_gather(n_tokens, chunk, 5)(table, ids)
    return out.reshape(B, L, DIM)


# speedup vs baseline: 5.3013x; 1.0235x over previous
import functools

import jax
import jax.numpy as jnp
from jax import lax
from jax.experimental import pallas as pl
from jax.experimental.pallas import tpu as pltpu
from jax.experimental.pallas import tpu_sc as plsc

DIM = 128
RANK = 64

NC = 2
NS = 16
NW = NC * NS


def _mm_body(u_ref, v_ref, e_ref):
    e_ref[...] = lax.dot_general(
        u_ref[...].astype(jnp.bfloat16), v_ref[...].astype(jnp.bfloat16),
        dimension_numbers=(((1,), (0,)), ((), ())),
        preferred_element_type=jnp.float32,
        precision=lax.Precision.DEFAULT,
    )


@functools.partial(jax.jit, static_argnames=("blk",))
def _compute_table(U, V, blk=10000):
    vocab = U.shape[0]
    return pl.pallas_call(
        _mm_body,
        grid=(vocab // blk,),
        in_specs=[
            pl.BlockSpec((blk, RANK), lambda i: (i, 0)),
            pl.BlockSpec((RANK, DIM), lambda i: (0, 0)),
        ],
        out_specs=pl.BlockSpec((blk, DIM), lambda i: (i, 0)),
        out_shape=jax.ShapeDtypeStruct((vocab, DIM), jnp.float32),
    )(U, V)


@functools.lru_cache(maxsize=None)
def _make_gather(n_tokens, chunk, nbuf):
    n_rows_idx = n_tokens // chunk
    n_chunks = n_rows_idx // NW
    assert n_tokens % (NW * chunk) == 0 and n_chunks % nbuf == 0
    n_outer = n_chunks // nbuf

    mesh = plsc.VectorSubcoreMesh(core_axis_name="c", subcore_axis_name="s")
    scratch = [
        pltpu.VMEM((n_chunks, chunk), jnp.int32),
        pltpu.VMEM((nbuf, chunk, DIM), jnp.float32),
    ]
    scratch += [pltpu.SemaphoreType.DMA] * (2 * nbuf)

    @functools.partial(
        pl.kernel,
        out_type=jax.ShapeDtypeStruct((n_tokens, DIM), jnp.float32),
        mesh=mesh,
        scratch_types=scratch,
    )
    def gather_kernel(table_hbm, idx_hbm, out_hbm, idx_v, rows_v, *sems):
        gs = sems[:nbuf]
        ws = sems[nbuf:]
        wid = lax.axis_index("s") * NC + lax.axis_index("c")
        rbase = wid * n_chunks
        tbase = rbase * chunk

        def start_gather(g, slot):
            pltpu.async_copy(table_hbm.at[idx_v.at[g]], rows_v.at[slot], gs[slot])

        def wait_gather(slot):
            pltpu.make_async_copy(
                table_hbm.at[idx_v.at[0]], rows_v.at[slot], gs[slot]
            ).wait()

        def start_write(g, slot):
            pltpu.async_copy(
                rows_v.at[slot], out_hbm.at[pl.ds(tbase + g * chunk, chunk)],
                ws[slot],
            )

        def wait_write(slot):
            pltpu.make_async_copy(
                rows_v.at[slot], out_hbm.at[pl.ds(tbase, chunk)], ws[slot]
            ).wait()

        pltpu.sync_copy(idx_hbm.at[wid], idx_v)
        for b in range(nbuf - 1):
            start_gather(b, b)

        def outer(j, carry):
            for b in range(nbuf):
                s = b
                sp = (b - 1) % nbuf
                g = j * nbuf + b
                gp = g + nbuf - 1
                if b == 0:
                    @pl.when(j > 0)
                    def _():
                        wait_write(sp)
                    start_gather(gp, sp)
                else:
                    @pl.when(j < n_outer - 1)
                    def _():
                        wait_write(sp)
                        start_gather(gp, sp)
                wait_gather(s)
                start_write(g, s)
            return carry

        lax.fori_loop(0, n_outer, outer, 0)
        for s in range(nbuf):
            wait_write(s)

    return gather_kernel


def kernel(token_ids, U, V):
    B, L = token_ids.shape
    n_tokens = B * L
    chunk = 128
    ids = token_ids.reshape(NW, n_tokens // (NW * chunk), chunk).astype(jnp.int32)
    table = _compute_table(U, V)
    out = _make_gather(n_tokens, chunk, 5)(table, ids)
    return out.reshape(B, L, DIM)


# speedup vs baseline: 7.2803x; 1.3733x over previous
import functools

import jax
import jax.numpy as jnp
from jax import lax
from jax.experimental import pallas as pl
from jax.experimental.pallas import tpu as pltpu
from jax.experimental.pallas import tpu_sc as plsc

DIM = 128
RANK = 64

NC = 2
NS = 16
NW = NC * NS


def _mm_body(ut_ref, v_ref, e_ref):
    e_ref[...] = lax.dot_general(
        ut_ref[...].astype(jnp.bfloat16), v_ref[...].astype(jnp.bfloat16),
        dimension_numbers=(((0,), (0,)), ((), ())),
        preferred_element_type=jnp.float32,
        precision=lax.Precision.DEFAULT,
    )


@functools.partial(jax.jit, static_argnames=("blk",))
def _compute_table(U, V, blk=14336):
    vocab = U.shape[0]
    vocab_pad = ((vocab + blk - 1) // blk) * blk
    return pl.pallas_call(
        _mm_body,
        grid=(vocab_pad // blk,),
        in_specs=[
            pl.BlockSpec((RANK, blk), lambda i: (0, i)),
            pl.BlockSpec((RANK, DIM), lambda i: (0, 0)),
        ],
        out_specs=pl.BlockSpec((blk, DIM), lambda i: (i, 0)),
        out_shape=jax.ShapeDtypeStruct((vocab_pad, DIM), jnp.float32),
    )(U.T, V)


@functools.lru_cache(maxsize=None)
def _make_gather(n_tokens, chunk, nbuf):
    n_rows_idx = n_tokens // chunk
    n_chunks = n_rows_idx // NW
    assert n_tokens % (NW * chunk) == 0 and n_chunks % nbuf == 0
    n_outer = n_chunks // nbuf

    mesh = plsc.VectorSubcoreMesh(core_axis_name="c", subcore_axis_name="s")
    scratch = [
        pltpu.VMEM((n_chunks, chunk), jnp.int32),
        pltpu.VMEM((nbuf, chunk, DIM), jnp.float32),
    ]
    scratch += [pltpu.SemaphoreType.DMA] * (2 * nbuf)

    @functools.partial(
        pl.kernel,
        out_type=jax.ShapeDtypeStruct((n_tokens, DIM), jnp.float32),
        mesh=mesh,
        scratch_types=scratch,
    )
    def gather_kernel(table_hbm, idx_hbm, out_hbm, idx_v, rows_v, *sems):
        gs = sems[:nbuf]
        ws = sems[nbuf:]
        wid = lax.axis_index("s") * NC + lax.axis_index("c")
        rbase = wid * n_chunks
        tbase = rbase * chunk

        def start_gather(g, slot):
            pltpu.async_copy(table_hbm.at[idx_v.at[g]], rows_v.at[slot], gs[slot])

        def wait_gather(slot):
            pltpu.make_async_copy(
                table_hbm.at[idx_v.at[0]], rows_v.at[slot], gs[slot]
            ).wait()

        def start_write(g, slot):
            pltpu.async_copy(
                rows_v.at[slot], out_hbm.at[pl.ds(tbase + g * chunk, chunk)],
                ws[slot],
            )

        def wait_write(slot):
            pltpu.make_async_copy(
                rows_v.at[slot], out_hbm.at[pl.ds(tbase, chunk)], ws[slot]
            ).wait()

        pltpu.sync_copy(idx_hbm.at[wid], idx_v)
        for b in range(nbuf - 1):
            start_gather(b, b)

        def outer(j, carry):
            for b in range(nbuf):
                s = b
                sp = (b - 1) % nbuf
                g = j * nbuf + b
                gp = g + nbuf - 1
                if b == 0:
                    @pl.when(j > 0)
                    def _():
                        wait_write(sp)
                    start_gather(gp, sp)
                else:
                    @pl.when(j < n_outer - 1)
                    def _():
                        wait_write(sp)
                        start_gather(gp, sp)
                wait_gather(s)
                start_write(g, s)
            return carry

        lax.fori_loop(0, n_outer, outer, 0)
        for s in range(nbuf):
            wait_write(s)

    return gather_kernel


def kernel(token_ids, U, V):
    B, L = token_ids.shape
    n_tokens = B * L
    chunk = 128
    ids = token_ids.reshape(NW, n_tokens // (NW * chunk), chunk).astype(jnp.int32)
    table = _compute_table(U, V)
    out = _make_gather(n_tokens, chunk, 5)(table, ids)
    return out.reshape(B, L, DIM)


# speedup vs baseline: 7.3538x; 1.0101x over previous
import functools

import jax
import jax.numpy as jnp
from jax import lax
from jax.experimental import pallas as pl
from jax.experimental.pallas import tpu as pltpu
from jax.experimental.pallas import tpu_sc as plsc

DIM = 128
RANK = 64

NC = 2
NS = 16
NW = NC * NS


def _mm_body(ut_ref, v_ref, e_ref):
    e_ref[...] = lax.dot_general(
        ut_ref[...].astype(jnp.bfloat16), v_ref[...].astype(jnp.bfloat16),
        dimension_numbers=(((0,), (0,)), ((), ())),
        preferred_element_type=jnp.float32,
        precision=lax.Precision.DEFAULT,
    )


@functools.partial(jax.jit, static_argnames=("blk",))
def _compute_table(U, V, blk=14336):
    vocab = U.shape[0]
    vocab_pad = ((vocab + blk - 1) // blk) * blk
    return pl.pallas_call(
        _mm_body,
        grid=(vocab_pad // blk,),
        in_specs=[
            pl.BlockSpec((RANK, blk), lambda i: (0, i)),
            pl.BlockSpec((RANK, DIM), lambda i: (0, 0)),
        ],
        out_specs=pl.BlockSpec((blk, DIM), lambda i: (i, 0)),
        out_shape=jax.ShapeDtypeStruct((vocab_pad, DIM), jnp.float32),
    )(U.T, V)


@functools.lru_cache(maxsize=None)
def _make_gather(n_tokens, chunk, nbuf):
    n_rows_idx = n_tokens // chunk
    n_chunks = n_rows_idx // NW
    assert n_tokens % (NW * chunk) == 0 and n_chunks % nbuf == 0
    n_outer = n_chunks // nbuf

    mesh = plsc.VectorSubcoreMesh(core_axis_name="c", subcore_axis_name="s")
    scratch = [
        pltpu.VMEM((n_chunks, chunk), jnp.int32),
        pltpu.VMEM((nbuf, chunk, DIM), jnp.float32),
    ]
    scratch += [pltpu.SemaphoreType.DMA] * (2 * nbuf)

    @functools.partial(
        pl.kernel,
        out_type=jax.ShapeDtypeStruct((n_tokens, DIM), jnp.float32),
        mesh=mesh,
        scratch_types=scratch,
    )
    def gather_kernel(table_hbm, idx_hbm, out_hbm, idx_v, rows_v, *sems):
        gs = sems[:nbuf]
        ws = sems[nbuf:]
        wid = lax.axis_index("s") * NC + lax.axis_index("c")
        rbase = wid * n_chunks
        tbase = rbase * chunk

        def start_gather(g, slot):
            pltpu.async_copy(table_hbm.at[idx_v.at[g]], rows_v.at[slot], gs[slot])

        def wait_gather(slot):
            pltpu.make_async_copy(
                table_hbm.at[idx_v.at[0]], rows_v.at[slot], gs[slot]
            ).wait()

        def start_write(g, slot):
            pltpu.async_copy(
                rows_v.at[slot], out_hbm.at[pl.ds(tbase + g * chunk, chunk)],
                ws[slot],
            )

        def wait_write(slot):
            pltpu.make_async_copy(
                rows_v.at[slot], out_hbm.at[pl.ds(tbase, chunk)], ws[slot]
            ).wait()

        pltpu.sync_copy(idx_hbm.at[wid], idx_v)
        for b in range(nbuf - 1):
            start_gather(b, b)

        def outer(j, carry):
            for b in range(nbuf):
                s = b
                sp = (b - 1) % nbuf
                g = j * nbuf + b
                gp = g + nbuf - 1
                if b == 0:
                    @pl.when(j > 0)
                    def _():
                        wait_write(sp)
                    start_gather(gp, sp)
                else:
                    @pl.when(j < n_outer - 1)
                    def _():
                        wait_write(sp)
                        start_gather(gp, sp)
                wait_gather(s)
                start_write(g, s)
            return carry

        lax.fori_loop(0, n_outer, outer, 0)
        for s in range(nbuf):
            wait_write(s)

    return gather_kernel


def kernel(token_ids, U, V):
    B, L = token_ids.shape
    n_tokens = B * L
    chunk = 80
    ids = token_ids.reshape(NW, n_tokens // (NW * chunk), chunk).astype(jnp.int32)
    table = _compute_table(U, V)
    out = _make_gather(n_tokens, chunk, 10)(table, ids)
    return out.reshape(B, L, DIM)
